# baseline (speedup 1.0000x reference)
import functools

import jax
import jax.numpy as jnp
import numpy as np
from jax.experimental import pallas as pl
from jax.experimental.pallas import tpu as pltpu

_N = 10000
_H = 100
_K = 30
_DT = 0.5
_QB = 200
_BIG = np.float32(3.4e38)


def _fused_kernel(u0c_ref, s0c_ref, e1c_ref, e2c_ref,
                  u0r_ref, s0r_ref, e1r_ref, e2r_ref,
                  scal_ref, W1_ref, b1_ref, W2t_ref,
                  cost_ref, u1_ref, s1_ref, al_ref, be_ref, ga_ref):
    i = pl.program_id(0)
    u0q = u0c_ref[...]
    s0q = s0c_ref[...]
    e1q = e1c_ref[...]
    e2q = e2c_ref[...]
    e1k = e1r_ref[...]
    e2k = e2r_ref[...]

    h = jnp.maximum(u0q * W1_ref[0:1, :] + s0q * W1_ref[1:2, :] + b1_ref[...],
                    0.0)
    o0 = jax.nn.sigmoid(jnp.sum(h * W2t_ref[0:1, :], axis=1, keepdims=True)
                        + scal_ref[3])
    o1 = jax.nn.sigmoid(jnp.sum(h * W2t_ref[1:2, :], axis=1, keepdims=True)
                        + scal_ref[4])
    o2 = jax.nn.sigmoid(jnp.sum(h * W2t_ref[2:3, :], axis=1, keepdims=True)
                        + scal_ref[5])
    alphas = o0 * scal_ref[0]
    beta = o1 * scal_ref[1]
    gamma = o2 * scal_ref[2]
    uv = (alphas - beta * u0q) * _DT
    sv = (beta * u0q - gamma * s0q) * _DT
    u1_ref[...] = u0q + uv
    s1_ref[...] = s0q + sv
    al_ref[...] = alphas
    be_ref[...] = beta
    ga_ref[...] = gamma

    sqq = e1q * e1q + e2q * e2q
    sqk = e1k * e1k + e2k * e2k
    e1qb = e1q.astype(jnp.bfloat16).astype(jnp.float32)
    e2qb = e2q.astype(jnp.bfloat16).astype(jnp.float32)
    e1kb = e1k.astype(jnp.bfloat16).astype(jnp.float32)
    e2kb = e2k.astype(jnp.bfloat16).astype(jnp.float32)
    d2 = sqq + sqk - 2.0 * (e1qb * e1kb + e2qb * e2kb)

    t0 = jnp.min(d2, axis=1, keepdims=True)
    max_sqk = jnp.max(sqk, axis=1, keepdims=True)
    r = jnp.sqrt(sqq) + jnp.sqrt(max_sqk)
    hi0 = r * r * np.float32(1.05) + np.float32(1.0)
    done0 = jnp.zeros_like(t0, dtype=jnp.bool_)

    ones_n = jnp.ones((d2.shape[1], 1), jnp.float32)
    lo, hi, done, t = t0, hi0, done0, t0
    for _ in range(12):
        lo_eff = jnp.maximum(lo, hi * np.float32(1e-9))
        mid = jnp.sqrt(lo_eff * hi)
        sel01 = jnp.where(d2 <= mid, 1.0, 0.0)
        c = jnp.dot(sel01, ones_n,
                    preferred_element_type=jnp.float32)
        hit = (c == np.float32(_K)) & (~done)
        t = jnp.where(hit, mid, t)
        done = done | hit
        below = c < np.float32(_K)
        lo = jnp.where(done | ~below, lo, mid)
        hi = jnp.where(done | below, hi, mid)
    t30 = jnp.where(done, t, hi)

    col = jax.lax.broadcasted_iota(jnp.int32, d2.shape, 1)
    minidx = jnp.min(jnp.where(d2 <= t0, col, np.int32(2**31 - 1)),
                     axis=1, keepdims=True)

    unv = u0r_ref[...] - u0q
    snv = s0r_ref[...] - s0q
    nv2 = unv * unv + snv * snv
    v2 = uv * uv + sv * sv
    sqv2 = jnp.sqrt(jnp.maximum(v2, 1e-30))
    score = jnp.where(nv2 > 0.0,
                      (unv * uv + snv * sv)
                      * jax.lax.rsqrt(jnp.maximum(nv2, 1e-30)),
                      sqv2)
    sel = (d2 <= t30) & (col != minidx)
    smax = jnp.max(jnp.where(sel, score, -_BIG), axis=1, keepdims=True)
    cmax = jnp.where(v2 > 0.0, smax / sqv2, 1.0)
    bs = jnp.sum(1.0 - cmax, axis=0, keepdims=True) * np.float32(1.0 / _N)

    @pl.when(i == 0)
    def _():
        cost_ref[...] = jnp.zeros_like(cost_ref)
    cost_ref[...] += bs


def kernel(u0, s0, alpha0, beta0, gamma0, embedding1, embedding2, epoch_num,
           W1, b1, W2, b2):
    del epoch_num
    n = u0.shape[0]
    grid = n // _QB
    col = lambda x: x.reshape(n, 1)
    row = lambda x: x.reshape(1, n)
    scal = jnp.concatenate([alpha0, beta0, gamma0, b2])

    qspec = pl.BlockSpec((_QB, 1), lambda i: (i, 0))
    rspec = pl.BlockSpec((1, n), lambda i: (0, 0))
    full = lambda s: pl.BlockSpec(s, lambda i: (0, 0))

    out = pl.pallas_call(
        _fused_kernel,
        grid=(grid,),
        in_specs=[qspec, qspec, qspec, qspec,
                  rspec, rspec, rspec, rspec,
                  pl.BlockSpec(memory_space=pltpu.SMEM),
                  full(W1.shape), full((1, _H)), full((3, _H))],
        out_specs=[pl.BlockSpec((1, 1), lambda i: (0, 0)),
                   qspec, qspec, qspec, qspec, qspec],
        out_shape=[jax.ShapeDtypeStruct((1, 1), jnp.float32)] +
                  [jax.ShapeDtypeStruct((n, 1), jnp.float32)] * 5,
    )(col(u0), col(s0), col(embedding1), col(embedding2),
      row(u0), row(s0), row(embedding1), row(embedding2),
      scal, W1, b1.reshape(1, _H), W2.T, )

    cost, u1, s1, alphas, beta, gamma = out
    flat = lambda x: x.reshape(n)
    return (cost[0, 0], flat(u1), flat(s1), flat(alphas), flat(beta),
            flat(gamma))


# speedup vs baseline: 1.6340x; 1.6340x over previous
import jax
import jax.numpy as jnp
import numpy as np
from jax.experimental import pallas as pl
from jax.experimental.pallas import tpu as pltpu

_N = 10000
_H = 100
_K = 30
_DT = 0.5
_QB = 200
_BIG = np.float32(3.4e38)


def _fused_kernel(u0c_ref, s0c_ref, e1c_ref, e2c_ref,
                  u0r_ref, s0r_ref, e1r_ref, e2r_ref,
                  scal_ref, W1_ref, b1_ref, W2t_ref,
                  cost_ref, u1_ref, s1_ref, al_ref, be_ref, ga_ref):
    i = pl.program_id(0)
    u0q = u0c_ref[...]
    s0q = s0c_ref[...]
    e1q = e1c_ref[...]
    e2q = e2c_ref[...]
    e1k = e1r_ref[...]
    e2k = e2r_ref[...]

    h = jnp.maximum(u0q * W1_ref[0:1, :] + s0q * W1_ref[1:2, :] + b1_ref[...],
                    0.0)
    o0 = jax.nn.sigmoid(jnp.sum(h * W2t_ref[0:1, :], axis=1, keepdims=True)
                        + scal_ref[3])
    o1 = jax.nn.sigmoid(jnp.sum(h * W2t_ref[1:2, :], axis=1, keepdims=True)
                        + scal_ref[4])
    o2 = jax.nn.sigmoid(jnp.sum(h * W2t_ref[2:3, :], axis=1, keepdims=True)
                        + scal_ref[5])
    alphas = o0 * scal_ref[0]
    beta = o1 * scal_ref[1]
    gamma = o2 * scal_ref[2]
    uv = (alphas - beta * u0q) * _DT
    sv = (beta * u0q - gamma * s0q) * _DT
    u1_ref[...] = u0q + uv
    s1_ref[...] = s0q + sv
    al_ref[...] = alphas
    be_ref[...] = beta
    ga_ref[...] = gamma

    sqq = e1q * e1q + e2q * e2q
    sqk = e1k * e1k + e2k * e2k
    e1qb = e1q.astype(jnp.bfloat16).astype(jnp.float32)
    e2qb = e2q.astype(jnp.bfloat16).astype(jnp.float32)
    e1kb = e1k.astype(jnp.bfloat16).astype(jnp.float32)
    e2kb = e2k.astype(jnp.bfloat16).astype(jnp.float32)
    d2 = sqq + sqk - 2.0 * (e1qb * e1kb + e2qb * e2kb)

    t0 = jnp.min(d2, axis=1, keepdims=True)
    max_sqk = jnp.max(sqk, axis=1, keepdims=True)
    r = jnp.sqrt(sqq) + jnp.sqrt(max_sqk)
    hi0 = r * r * np.float32(1.05) + np.float32(1.0)
    done0 = jnp.zeros_like(t0, dtype=jnp.bool_)

    lo, hi, done, t = t0, hi0, done0, t0
    for _ in range(12):
        lo_eff = jnp.maximum(lo, hi * np.float32(1e-9))
        mid = jnp.sqrt(lo_eff * hi)
        c = jnp.sum(jnp.where(d2 <= mid, 1.0, 0.0), axis=1, keepdims=True)
        hit = (c == np.float32(_K)) & (~done)
        t = jnp.where(hit, mid, t)
        done = done | hit
        below = c < np.float32(_K)
        lo = jnp.where(done | ~below, lo, mid)
        hi = jnp.where(done | below, hi, mid)
    t30 = jnp.where(done, t, hi)

    col = jax.lax.broadcasted_iota(jnp.int32, d2.shape, 1)
    minidx = jnp.min(jnp.where(d2 <= t0, col, np.int32(2**31 - 1)),
                     axis=1, keepdims=True)

    unv = u0r_ref[...] - u0q
    snv = s0r_ref[...] - s0q
    nv2 = unv * unv + snv * snv
    v2 = uv * uv + sv * sv
    sqv2 = jnp.sqrt(jnp.maximum(v2, 1e-30))
    score = jnp.where(nv2 > 0.0,
                      (unv * uv + snv * sv)
                      * jax.lax.rsqrt(jnp.maximum(nv2, 1e-30)),
                      sqv2)
    sel = (d2 <= t30) & (col != minidx)
    smax = jnp.max(jnp.where(sel, score, -_BIG), axis=1, keepdims=True)
    cmax = jnp.where(v2 > 0.0, smax / sqv2, 1.0)
    bs = jnp.sum(1.0 - cmax, axis=0, keepdims=True) * np.float32(1.0 / _N)

    @pl.when(i == 0)
    def _():
        cost_ref[...] = jnp.zeros_like(cost_ref)
    cost_ref[...] += bs


def kernel(u0, s0, alpha0, beta0, gamma0, embedding1, embedding2, epoch_num,
           W1, b1, W2, b2):
    del epoch_num
    n = u0.shape[0]
    grid = n // _QB
    col = lambda x: x.reshape(n, 1)
    row = lambda x: x.reshape(1, n)
    scal = jnp.concatenate([alpha0, beta0, gamma0, b2])

    qspec = pl.BlockSpec((_QB, 1), lambda i: (i, 0))
    rspec = pl.BlockSpec((1, n), lambda i: (0, 0))
    full = lambda s: pl.BlockSpec(s, lambda i: (0, 0))

    out = pl.pallas_call(
        _fused_kernel,
        grid=(grid,),
        in_specs=[qspec, qspec, qspec, qspec,
                  rspec, rspec, rspec, rspec,
                  pl.BlockSpec(memory_space=pltpu.SMEM),
                  full(W1.shape), full((1, _H)), full((3, _H))],
        out_specs=[pl.BlockSpec((1, 1), lambda i: (0, 0)),
                   qspec, qspec, qspec, qspec, qspec],
        out_shape=[jax.ShapeDtypeStruct((1, 1), jnp.float32)] +
                  [jax.ShapeDtypeStruct((n, 1), jnp.float32)] * 5,
    )(col(u0), col(s0), col(embedding1), col(embedding2),
      row(u0), row(s0), row(embedding1), row(embedding2),
      scal, W1, b1.reshape(1, _H), W2.T, )

    cost, u1, s1, alphas, beta, gamma = out
    flat = lambda x: x.reshape(n)
    return (cost[0, 0], flat(u1), flat(s1), flat(alphas), flat(beta),
            flat(gamma))
